# Initial kernel scaffold; baseline (speedup 1.0000x reference)
import jax
import jax.numpy as jnp
from jax.experimental import pallas as pl


def kernel(feature_maps, init_potential_anchor):
    raise NotImplementedError("write your pallas kernel here")


# speedup vs baseline: 1.1445x; 1.1445x over previous
import functools

import jax
import jax.numpy as jnp
from jax import lax
from jax.experimental import pallas as pl
from jax.experimental.pallas import tpu as pltpu
from jax.experimental.pallas import tpu_sc as plsc

L = 16


def _make_sc_interp(B, C, H, W, N):
    HW = H * W
    total = B * N
    mesh = plsc.VectorSubcoreMesh(core_axis_name="c", subcore_axis_name="s")
    NW = mesh.num_cores * mesh.num_subcores
    chunks_per_w = -(-total // (NW * L))
    per_w = chunks_per_w * L
    npad = per_w * NW
    cslices = C // L

    @functools.partial(
        pl.kernel,
        out_type=jax.ShapeDtypeStruct((total, C), jnp.float32),
        mesh=mesh,
        scratch_types=[
            pltpu.VMEM((per_w,), jnp.float32),
            pltpu.VMEM((per_w,), jnp.float32),
            pltpu.VMEM((4 * L,), jnp.int32),
            pltpu.VMEM((4 * L, C), jnp.float32),
            pltpu.VMEM((L, C), jnp.float32),
            pltpu.SemaphoreType.DMA,
        ],
    )
    def sc_interp(table_hbm, xs_hbm, ys_hbm, out_hbm,
                  xs_v, ys_v, idx_v, rows_v, out_v, sem):
        wid = lax.axis_index("s") * mesh.num_cores + lax.axis_index("c")
        base = pl.multiple_of(wid * per_w, L)
        pltpu.sync_copy(xs_hbm.at[pl.ds(base, per_w)], xs_v)
        pltpu.sync_copy(ys_hbm.at[pl.ds(base, per_w)], ys_v)
        lane = lax.iota(jnp.int32, L)

        def chunk_body(ci, _):
            cbase = base + ci * L

            @pl.when(cbase < total)
            def _():
                off = pl.multiple_of(ci * L, L)
                px = xs_v[pl.ds(off, L)]
                py = ys_v[pl.ds(off, L)]
                x = jnp.minimum(jnp.maximum(px * (H - 1), 0.0), H - 1)
                y = jnp.minimum(jnp.maximum(py * (H - 1), 0.0), H - 1)
                xi = x.astype(jnp.int32)
                yi = y.astype(jnp.int32)
                wx = x - xi.astype(jnp.float32)
                wy = y - yi.astype(jnp.float32)
                xc = jnp.where(wx > 0.0, xi + 1, xi)
                yc = jnp.where(wy > 0.0, yi + 1, yi)
                p = cbase + lane
                badd = jnp.minimum(lax.div(p, N), B - 1) * HW
                row_t = badd + yi * W
                row_b = badd + yc * W
                idx_v[pl.ds(0, L)] = row_t + xi
                idx_v[pl.ds(L, L)] = row_t + xc
                idx_v[pl.ds(2 * L, L)] = row_b + xi
                idx_v[pl.ds(3 * L, L)] = row_b + xc
                pltpu.async_copy(table_hbm.at[idx_v], rows_v, sem).wait()

                for i in range(L):
                    wxs = wx[i]
                    wys = wy[i]

                    def cs_body(j, _, i=i, wxs=wxs, wys=wys):
                        sl = pl.ds(j * L, L)
                        lt = rows_v[i, sl]
                        rt = rows_v[L + i, sl]
                        lb = rows_v[2 * L + i, sl]
                        rb = rows_v[3 * L + i, sl]
                        t = lt + (rt - lt) * wxs
                        bt = lb + (rb - lb) * wxs
                        out_v[i, sl] = t + (bt - t) * wys
                        return 0

                    lax.fori_loop(0, cslices, cs_body, 0)
                pltpu.sync_copy(out_v, out_hbm.at[pl.ds(pl.multiple_of(cbase, L), L)])

            return 0

        lax.fori_loop(0, chunks_per_w, chunk_body, 0)

    return sc_interp, npad


def kernel(feature_maps, init_potential_anchor):
    B, C, H, W = feature_maps.shape
    N = init_potential_anchor.shape[1]
    sc_interp, npad = _make_sc_interp(B, C, H, W, N)
    table = feature_maps.transpose(0, 2, 3, 1).reshape(B * H * W, C)
    pa = init_potential_anchor.reshape(B * N, 2)
    pad = npad - B * N
    xs = jnp.pad(pa[:, 0], (0, pad))
    ys = jnp.pad(pa[:, 1], (0, pad))
    out = sc_interp(table, xs, ys)
    return out.reshape(B, N, C)


# speedup vs baseline: 1.3818x; 1.2073x over previous
import functools

import jax
import jax.numpy as jnp
from jax import lax
from jax.experimental import pallas as pl
from jax.experimental.pallas import tpu as pltpu
from jax.experimental.pallas import tpu_sc as plsc

L = 16


def _make_sc_interp(B, C, H, W, N):
    HW = H * W
    total = B * N
    mesh = plsc.VectorSubcoreMesh(core_axis_name="c", subcore_axis_name="s")
    NW = mesh.num_cores * mesh.num_subcores
    nchunk = -(-total // (NW * L))
    per_w = nchunk * L
    npad = per_w * NW
    cslices = C // L

    @functools.partial(
        pl.kernel,
        out_type=jax.ShapeDtypeStruct((total, C), jnp.float32),
        mesh=mesh,
        scratch_types=[
            pltpu.VMEM((per_w,), jnp.float32),
            pltpu.VMEM((per_w,), jnp.float32),
            pltpu.VMEM((4 * L,), jnp.int32),
            pltpu.VMEM((4 * L, C), jnp.float32),
            pltpu.VMEM((L, C), jnp.float32),
            pltpu.SemaphoreType.DMA,
        ],
    )
    def sc_interp(table_hbm, xs_hbm, ys_hbm, out_hbm,
                  xs_v, ys_v, idx_v, rows_v, out_v, sem):
        wid = lax.axis_index("s") * mesh.num_cores + lax.axis_index("c")
        base = pl.multiple_of(wid * per_w, L)
        pltpu.sync_copy(xs_hbm.at[pl.ds(base, per_w)], xs_v)
        pltpu.sync_copy(ys_hbm.at[pl.ds(base, per_w)], ys_v)
        lane = lax.iota(jnp.int32, L)

        def chunk_body(ci, _):
            cbase = base + ci * L

            @pl.when(cbase < total)
            def _():
                off = pl.multiple_of(ci * L, L)
                px = xs_v[pl.ds(off, L)]
                py = ys_v[pl.ds(off, L)]
                x = jnp.minimum(jnp.maximum(px * (H - 1), 0.0), H - 1)
                y = jnp.minimum(jnp.maximum(py * (H - 1), 0.0), H - 1)
                xi = x.astype(jnp.int32)
                yi = y.astype(jnp.int32)
                wx = x - xi.astype(jnp.float32)
                wy = y - yi.astype(jnp.float32)
                xc = jnp.where(wx > 0.0, xi + 1, xi)
                yc = jnp.where(wy > 0.0, yi + 1, yi)
                p = cbase + lane
                badd = jnp.minimum(lax.div(p, N), B - 1) * HW
                row_t = badd + yi * W
                row_b = badd + yc * W
                idx_v[pl.ds(0, L)] = row_t + xi
                idx_v[pl.ds(L, L)] = row_t + xc
                idx_v[pl.ds(2 * L, L)] = row_b + xi
                idx_v[pl.ds(3 * L, L)] = row_b + xc
                pltpu.async_copy(table_hbm.at[idx_v], rows_v, sem).wait()

                for i in range(L):
                    wxs = wx[i]
                    wys = wy[i]

                    @plsc.parallel_loop(0, cslices, unroll=4)
                    def _(j, i=i, wxs=wxs, wys=wys):
                        sl = pl.ds(j * L, L)
                        lt = rows_v[i, sl]
                        rt = rows_v[L + i, sl]
                        lb = rows_v[2 * L + i, sl]
                        rb = rows_v[3 * L + i, sl]
                        t = lt + (rt - lt) * wxs
                        bt = lb + (rb - lb) * wxs
                        out_v[i, sl] = t + (bt - t) * wys

                pltpu.sync_copy(out_v, out_hbm.at[pl.ds(pl.multiple_of(cbase, L), L)])

            return 0

        lax.fori_loop(0, nchunk, chunk_body, 0)

    return sc_interp, npad


def kernel(feature_maps, init_potential_anchor):
    B, C, H, W = feature_maps.shape
    N = init_potential_anchor.shape[1]
    sc_interp, npad = _make_sc_interp(B, C, H, W, N)
    table = feature_maps.transpose(0, 2, 3, 1).reshape(B * H * W, C)
    pa = init_potential_anchor.reshape(B * N, 2)
    pad = npad - B * N
    xs = jnp.pad(pa[:, 0], (0, pad))
    ys = jnp.pad(pa[:, 1], (0, pad))
    out = sc_interp(table, xs, ys)
    return out.reshape(B, N, C)


# speedup vs baseline: 2.2423x; 1.6228x over previous
import functools

import jax
import jax.numpy as jnp
from jax import lax
from jax.experimental import pallas as pl
from jax.experimental.pallas import tpu as pltpu
from jax.experimental.pallas import tpu_sc as plsc

L = 16


def _make_sc_interp(B, C, H, W, N):
    HW = H * W
    total = B * N
    mesh = plsc.VectorSubcoreMesh(core_axis_name="c", subcore_axis_name="s")
    NW = mesh.num_cores * mesh.num_subcores
    nchunk = 2 * -(-total // (NW * L * 2))
    per_w = nchunk * L
    npad = per_w * NW
    cslices = C // L

    @functools.partial(
        pl.kernel,
        out_type=jax.ShapeDtypeStruct((total, C), jnp.float32),
        mesh=mesh,
        scratch_types=[
            pltpu.VMEM((per_w,), jnp.float32),
            pltpu.VMEM((per_w,), jnp.float32),
            pltpu.VMEM((2, 4 * L), jnp.int32),
            pltpu.VMEM((2, 4 * L, C), jnp.float32),
            pltpu.VMEM((2, L, C), jnp.float32),
            pltpu.SemaphoreType.DMA,
            pltpu.SemaphoreType.DMA,
            pltpu.SemaphoreType.DMA,
            pltpu.SemaphoreType.DMA,
        ],
    )
    def sc_interp(table_hbm, xs_hbm, ys_hbm, out_hbm,
                  xs_v, ys_v, idx_v, rows_v, out_v, gsem0, gsem1, wsem0, wsem1):
        gsems = (gsem0, gsem1)
        wsems = (wsem0, wsem1)
        wid = lax.axis_index("s") * mesh.num_cores + lax.axis_index("c")
        base = pl.multiple_of(wid * per_w, L)
        pltpu.sync_copy(xs_hbm.at[pl.ds(base, per_w)], xs_v)
        pltpu.sync_copy(ys_hbm.at[pl.ds(base, per_w)], ys_v)
        lane = lax.iota(jnp.int32, L)

        def issue_gather(ci, buf):
            off = pl.multiple_of(ci * L, L)
            px = xs_v[pl.ds(off, L)]
            py = ys_v[pl.ds(off, L)]
            x = jnp.minimum(jnp.maximum(px * (H - 1), 0.0), H - 1)
            y = jnp.minimum(jnp.maximum(py * (H - 1), 0.0), H - 1)
            xi = x.astype(jnp.int32)
            yi = y.astype(jnp.int32)
            xc = jnp.where(x - xi.astype(jnp.float32) > 0.0, xi + 1, xi)
            yc = jnp.where(y - yi.astype(jnp.float32) > 0.0, yi + 1, yi)
            p = base + ci * L + lane
            badd = jnp.minimum(lax.div(p, N), B - 1) * HW
            row_t = badd + yi * W
            row_b = badd + yc * W
            idx_v[buf, pl.ds(0, L)] = row_t + xi
            idx_v[buf, pl.ds(L, L)] = row_t + xc
            idx_v[buf, pl.ds(2 * L, L)] = row_b + xi
            idx_v[buf, pl.ds(3 * L, L)] = row_b + xc
            pltpu.async_copy(table_hbm.at[idx_v.at[buf]], rows_v.at[buf], gsems[buf])

        def wait_gather(buf):
            pltpu.make_async_copy(
                table_hbm.at[idx_v.at[buf]], rows_v.at[buf], gsems[buf]).wait()

        def out_slot(ci):
            return out_hbm.at[pl.ds(pl.multiple_of(base + ci * L, L), L)]

        def valid(ci):
            return base + ci * L < total

        @pl.when(valid(0))
        def _():
            issue_gather(0, 0)

        def process(ci, buf):
            @pl.when((ci >= 2) & valid(ci - 2))
            def _():
                pltpu.make_async_copy(out_v.at[buf], out_slot(ci - 2), wsems[buf]).wait()

            @pl.when((ci + 1 < nchunk) & valid(ci + 1))
            def _():
                issue_gather(ci + 1, 1 - buf)

            @pl.when(valid(ci))
            def _():
                wait_gather(buf)
                off = pl.multiple_of(ci * L, L)
                px = xs_v[pl.ds(off, L)]
                py = ys_v[pl.ds(off, L)]
                x = jnp.minimum(jnp.maximum(px * (H - 1), 0.0), H - 1)
                y = jnp.minimum(jnp.maximum(py * (H - 1), 0.0), H - 1)
                wx = x - x.astype(jnp.int32).astype(jnp.float32)
                wy = y - y.astype(jnp.int32).astype(jnp.float32)
                for i in range(L):
                    wxs = wx[i]
                    wys = wy[i]

                    @plsc.parallel_loop(0, cslices, unroll=4)
                    def _(j, i=i, buf=buf, wxs=wxs, wys=wys):
                        sl = pl.ds(j * L, L)
                        lt = rows_v[buf, i, sl]
                        rt = rows_v[buf, L + i, sl]
                        lb = rows_v[buf, 2 * L + i, sl]
                        rb = rows_v[buf, 3 * L + i, sl]
                        t = lt + (rt - lt) * wxs
                        bt = lb + (rb - lb) * wxs
                        out_v[buf, i, sl] = t + (bt - t) * wys

                pltpu.async_copy(out_v.at[buf], out_slot(ci), wsems[buf])

        def pair_body(k, _):
            process(2 * k, 0)
            process(2 * k + 1, 1)
            return 0

        lax.fori_loop(0, nchunk // 2, pair_body, 0)

        for tail in (nchunk - 2, nchunk - 1):
            @pl.when(valid(tail))
            def _(tail=tail):
                pltpu.make_async_copy(
                    out_v.at[tail % 2], out_slot(tail), wsems[tail % 2]).wait()

    return sc_interp, npad


def kernel(feature_maps, init_potential_anchor):
    B, C, H, W = feature_maps.shape
    N = init_potential_anchor.shape[1]
    sc_interp, npad = _make_sc_interp(B, C, H, W, N)
    table = feature_maps.transpose(0, 2, 3, 1).reshape(B * H * W, C)
    pa = init_potential_anchor.reshape(B * N, 2)
    pad = npad - B * N
    xs = jnp.pad(pa[:, 0], (0, pad))
    ys = jnp.pad(pa[:, 1], (0, pad))
    out = sc_interp(table, xs, ys)
    return out.reshape(B, N, C)
